# baseline (device time: 54568 ns/iter reference)
import jax
import jax.numpy as jnp
from jax import lax
from jax.experimental import pallas as pl
from jax.experimental.pallas import tpu as pltpu

R = 256
MAXC = 2048 // R


def kernel(x, dest):
    m, n = x.shape
    my_y = lax.axis_index("y")

    iota = jnp.arange(m, dtype=jnp.int32)
    is0 = dest == 0
    cum = jnp.cumsum(is0.astype(jnp.int32))
    c0 = cum[m - 1]
    pos_in_group = jnp.where(is0, cum - 1, iota - cum)
    is_send = jnp.where(my_y == 0, ~is0, is0)
    rc = jnp.where(my_y == 0, m - c0, c0)
    slot = jnp.where(is_send, pos_in_group, rc + pos_in_group)
    onehot = (jnp.arange(m, dtype=jnp.int32)[:, None] == slot[None, :]).astype(
        jnp.bfloat16
    )

    def body(
        c0_ref, onehot_ref, x_ref, out_ref,
        xb_ref, xs_ref, recv_ref, send_sems, recv_sems,
    ):
        my_x = lax.axis_index("x")
        yy = lax.axis_index("y")
        peer = 1 - yy
        c0_ = c0_ref[0]
        c_keep = jnp.where(yy == 0, c0_, m - c0_)
        rc_ = m - c_keep
        n_chunks = (rc_ + R - 1) // R

        barrier_sem = pltpu.get_barrier_semaphore()
        pl.semaphore_signal(
            barrier_sem,
            inc=1,
            device_id=(my_x, peer),
            device_id_type=pl.DeviceIdType.MESH,
        )
        pl.semaphore_wait(barrier_sem, 1)

        xb_ref[...] = x_ref[...].astype(jnp.bfloat16)
        xb = xb_ref[...]

        for k in range(MAXC):
            blk = jnp.dot(
                onehot_ref[pl.ds(k * R, R), :],
                xb,
                preferred_element_type=jnp.float32,
            )
            xs_ref[pl.ds(k * R, R), :] = blk.astype(jnp.bfloat16)

            @pl.when(k < n_chunks)
            def _(k=k):
                pltpu.make_async_remote_copy(
                    src_ref=xs_ref.at[pl.ds(k * R, R)],
                    dst_ref=recv_ref.at[pl.ds(k * R, R)],
                    send_sem=send_sems.at[k],
                    recv_sem=recv_sems.at[k],
                    device_id=(my_x, peer),
                    device_id_type=pl.DeviceIdType.MESH,
                ).start()

        for k in range(MAXC):
            @pl.when(k < n_chunks)
            def _(k=k):
                rdma = pltpu.make_async_remote_copy(
                    src_ref=xs_ref.at[pl.ds(k * R, R)],
                    dst_ref=recv_ref.at[pl.ds(k * R, R)],
                    send_sem=send_sems.at[k],
                    recv_sem=recv_sems.at[k],
                    device_id=(my_x, peer),
                    device_id_type=pl.DeviceIdType.MESH,
                )
                rdma.wait_send()
                rdma.wait_recv()

        row = lax.broadcasted_iota(jnp.int32, (m, 1), 0)
        xs = xs_ref[...]
        recv = recv_ref[...]

        @pl.when(yy == 0)
        def _():
            out_ref[...] = pltpu.roll(
                jnp.where(row >= rc_, xs, recv), c_keep, 0
            )

        @pl.when(yy == 1)
        def _():
            out_ref[...] = jnp.where(row < rc_, recv, xs)

    return pl.pallas_call(
        body,
        out_shape=jax.ShapeDtypeStruct((m, n), jnp.bfloat16),
        in_specs=[
            pl.BlockSpec(memory_space=pltpu.SMEM),
            pl.BlockSpec(memory_space=pltpu.VMEM),
            pl.BlockSpec(memory_space=pltpu.VMEM),
        ],
        out_specs=pl.BlockSpec(memory_space=pltpu.VMEM),
        scratch_shapes=[
            pltpu.VMEM((m, n), jnp.bfloat16),
            pltpu.VMEM((m, n), jnp.bfloat16),
            pltpu.VMEM((m, n), jnp.bfloat16),
            pltpu.SemaphoreType.DMA((MAXC,)),
            pltpu.SemaphoreType.DMA((MAXC,)),
        ],
        compiler_params=pltpu.CompilerParams(
            collective_id=0, vmem_limit_bytes=64 * 1024 * 1024
        ),
    )(jnp.reshape(c0, (1,)), onehot, x)


# device time: 41646 ns/iter; 1.3103x vs baseline; 1.3103x over previous
import jax
import jax.numpy as jnp
from jax import lax
from jax.experimental import pallas as pl
from jax.experimental.pallas import tpu as pltpu

R = 256
MAXC = 2048 // R


def kernel(x, dest):
    m, n = x.shape
    my_y = lax.axis_index("y")

    iota = jnp.arange(m, dtype=jnp.int32)
    is0 = dest == 0
    cum = jnp.cumsum(is0.astype(jnp.int32))
    c0 = cum[m - 1]
    pos_in_group = jnp.where(is0, cum - 1, iota - cum)
    is_send = jnp.where(my_y == 0, ~is0, is0)
    rc = jnp.where(my_y == 0, m - c0, c0)
    slot = jnp.where(is_send, pos_in_group, rc + pos_in_group)

    def body(
        c0_ref, slot_ref, x_ref, out_ref,
        xb_ref, xs_ref, recv_ref, send_sems, recv_sems,
    ):
        my_x = lax.axis_index("x")
        yy = lax.axis_index("y")
        peer = 1 - yy
        c0_ = c0_ref[0]
        c_keep = jnp.where(yy == 0, c0_, m - c0_)
        rc_ = m - c_keep
        n_chunks = (rc_ + R - 1) // R

        barrier_sem = pltpu.get_barrier_semaphore()
        pl.semaphore_signal(
            barrier_sem,
            inc=1,
            device_id=(my_x, peer),
            device_id_type=pl.DeviceIdType.MESH,
        )
        pl.semaphore_wait(barrier_sem, 1)

        xb_ref[...] = x_ref[...].astype(jnp.bfloat16)
        xb = xb_ref[...]
        slot_row = slot_ref[...]

        for k in range(MAXC):
            pk = lax.broadcasted_iota(jnp.int32, (R, m), 0) + k * R
            onehot = (slot_row == pk).astype(jnp.bfloat16)
            blk = jnp.dot(onehot, xb, preferred_element_type=jnp.float32)
            xs_ref[pl.ds(k * R, R), :] = blk.astype(jnp.bfloat16)

            @pl.when(k < n_chunks)
            def _(k=k):
                pltpu.make_async_remote_copy(
                    src_ref=xs_ref.at[pl.ds(k * R, R)],
                    dst_ref=recv_ref.at[pl.ds(k * R, R)],
                    send_sem=send_sems.at[k],
                    recv_sem=recv_sems.at[k],
                    device_id=(my_x, peer),
                    device_id_type=pl.DeviceIdType.MESH,
                ).start()

        for k in range(MAXC):
            @pl.when(k < n_chunks)
            def _(k=k):
                rdma = pltpu.make_async_remote_copy(
                    src_ref=xs_ref.at[pl.ds(k * R, R)],
                    dst_ref=recv_ref.at[pl.ds(k * R, R)],
                    send_sem=send_sems.at[k],
                    recv_sem=recv_sems.at[k],
                    device_id=(my_x, peer),
                    device_id_type=pl.DeviceIdType.MESH,
                )
                rdma.wait_send()
                rdma.wait_recv()

        row = lax.broadcasted_iota(jnp.int32, (m, 1), 0)
        xs = xs_ref[...]
        recv = recv_ref[...]

        @pl.when(yy == 0)
        def _():
            out_ref[...] = pltpu.roll(
                jnp.where(row >= rc_, xs, recv), c_keep, 0
            )

        @pl.when(yy == 1)
        def _():
            out_ref[...] = jnp.where(row < rc_, recv, xs)

    return pl.pallas_call(
        body,
        out_shape=jax.ShapeDtypeStruct((m, n), jnp.bfloat16),
        in_specs=[
            pl.BlockSpec(memory_space=pltpu.SMEM),
            pl.BlockSpec(memory_space=pltpu.VMEM),
            pl.BlockSpec(memory_space=pltpu.VMEM),
        ],
        out_specs=pl.BlockSpec(memory_space=pltpu.VMEM),
        scratch_shapes=[
            pltpu.VMEM((m, n), jnp.bfloat16),
            pltpu.VMEM((m, n), jnp.bfloat16),
            pltpu.VMEM((m, n), jnp.bfloat16),
            pltpu.SemaphoreType.DMA((MAXC,)),
            pltpu.SemaphoreType.DMA((MAXC,)),
        ],
        compiler_params=pltpu.CompilerParams(
            collective_id=0, vmem_limit_bytes=64 * 1024 * 1024
        ),
    )(jnp.reshape(c0, (1,)), jnp.reshape(slot, (1, m)), x)
